# baseline (speedup 1.0000x reference)
import functools
import math

import jax
import jax.numpy as jnp
from jax import lax
from jax.experimental import pallas as pl
from jax.experimental.pallas import tpu as pltpu
from jax.experimental.pallas import tpu_sc as plsc

D_MODEL = 128
SCALE = math.sqrt(float(D_MODEL))
LANES = 16

NUM_CORES = 2
NUM_SUBCORES = 16
NUM_WORKERS = NUM_CORES * NUM_SUBCORES

N_BUF = 4


def _make_sc_gather(batch: int, chunk: int):
    assert batch % NUM_WORKERS == 0
    rows_per_w = batch // NUM_WORKERS
    assert rows_per_w % chunk == 0
    n_chunks = rows_per_w // chunk
    assert chunk % 8 == 0
    assert (n_chunks - 4) % N_BUF == 0 and n_chunks >= 2 * N_BUF

    mesh = plsc.VectorSubcoreMesh(
        core_axis_name="c", subcore_axis_name="s", num_cores=NUM_CORES
    )

    @functools.partial(
        pl.kernel,
        mesh=mesh,
        out_type=jax.ShapeDtypeStruct((batch, D_MODEL), jnp.float32),
        scratch_types=[
            pltpu.VMEM((rows_per_w,), jnp.int32),
        ]
        + [pltpu.VMEM((chunk, D_MODEL), jnp.float32)] * N_BUF
        + [pltpu.SemaphoreType.DMA] * (2 * N_BUF),
    )
    def emb_kernel(idx_hbm, lut_hbm, out_hbm, idx_v, *bufs_and_sems):
        bufs = bufs_and_sems[:N_BUF]
        gsems = bufs_and_sems[N_BUF : 2 * N_BUF]
        osems = bufs_and_sems[2 * N_BUF :]
        wid = lax.axis_index("s") * NUM_CORES + lax.axis_index("c")
        base = wid * rows_per_w
        pltpu.sync_copy(idx_hbm.at[pl.ds(base, rows_per_w)], idx_v)

        def start_gather(c, b):
            pltpu.async_copy(
                lut_hbm.at[idx_v.at[pl.ds(c * chunk, chunk)]], bufs[b],
                gsems[b],
            )

        def wait_gather(b):
            pltpu.make_async_copy(
                lut_hbm.at[idx_v.at[pl.ds(0, chunk)]], bufs[b], gsems[b]
            ).wait()

        def start_out(c, b):
            pltpu.async_copy(
                bufs[b], out_hbm.at[pl.ds(base + c * chunk, chunk)], osems[b]
            )

        def wait_out(b):
            pltpu.make_async_copy(
                bufs[b], out_hbm.at[pl.ds(base, chunk)], osems[b]
            ).wait()

        def scale_buf(b):
            def scale_body(r, carry):
                for j in range(D_MODEL // LANES):
                    sl = pl.ds(j * LANES, LANES)
                    bufs[b][r, sl] = bufs[b][r, sl] * SCALE
                return carry

            lax.fori_loop(0, chunk, scale_body, 0, unroll=4)

        ahead = 2
        for c in range(ahead):
            start_gather(c, c)
        for c in range(ahead):
            start_gather(c + ahead, c + ahead)
            wait_gather(c)
            scale_buf(c)
            start_out(c, c)

        def group(k, carry):
            c0 = ahead + k * N_BUF
            for j in range(N_BUF):
                c = c0 + j
                b = (ahead + j) % N_BUF
                nb = j % N_BUF
                wait_out(nb)
                start_gather(c + ahead, nb)
                wait_gather(b)
                scale_buf(b)
                start_out(c, b)
            return carry

        n_groups = (n_chunks - 2 * ahead) // N_BUF
        lax.fori_loop(0, n_groups, group, 0)

        for c in range(n_chunks - ahead, n_chunks):
            b = c % N_BUF
            wait_gather(b)
            scale_buf(b)
            start_out(c, b)
        for c in range(n_chunks - N_BUF, n_chunks):
            wait_out(c % N_BUF)

    return emb_kernel


def kernel(x, lut):
    b, s = x.shape
    batch = b * s
    idx = x.T.reshape(batch).astype(jnp.int32)
    fn = _make_sc_gather(batch, chunk=200)
    out = fn(idx, lut)
    return out.reshape(s, b, D_MODEL).transpose(1, 0, 2)
